# baseline (speedup 1.0000x reference)
import functools

import jax
import jax.numpy as jnp
from jax import lax
from jax.experimental import pallas as pl
from jax.experimental.pallas import tpu as pltpu
from jax.experimental.pallas import tpu_sc as plsc

NC = 2
NS = 16
NW = NC * NS
LANES = 16
CHUNK = 128
DEPTH = 6
K0_NUM = 64
K1_NUM = 94


def _mlp_deg_body(x_ref, wp_ref, bp_ref, wm_ref, bm_ref, degp_ref,
                  hid_ref, dinv_ref, g_ref):
    h = jnp.maximum(
        lax.dot_general(x_ref[...], wp_ref[...], (((1,), (1,)), ((), ())),
                        preferred_element_type=jnp.float32) + bp_ref[...],
        0.0)
    hid = lax.dot_general(
        h, wm_ref[...], (((1,), (1,)), ((), ())),
        preferred_element_type=jnp.float32) + bm_ref[...]
    hid_ref[...] = hid
    deg = 1.0 + jnp.sum(degp_ref[...], axis=0)
    deg_safe = jnp.where(deg > 0, deg, 1.0)
    dinv = jnp.where(deg > 0, lax.rsqrt(deg_safe), 0.0)
    dinv_ref[...] = dinv[:, None]
    g_ref[...] = dinv[:, None] * hid


def _interleave_bf16(g):
    n, d = g.shape
    g4 = g.reshape(n, d // 32, 2, 16)
    gi = jnp.stack([g4[:, :, 0, :], g4[:, :, 1, :]], axis=-1)
    return gi.reshape(n, d).astype(jnp.bfloat16)


def _mlp_deg(x, Wp, bp, Wm, bm, degp):
    n = x.shape[0]
    d = Wm.shape[0]
    return pl.pallas_call(
        _mlp_deg_body,
        out_shape=[
            jax.ShapeDtypeStruct((n, d), jnp.float32),
            jax.ShapeDtypeStruct((n, 1), jnp.float32),
            jax.ShapeDtypeStruct((n, d), jnp.float32),
        ],
    )(x, Wp, bp.reshape(1, -1), Wm, bm.reshape(1, -1), degp)


def _combine_body(t_ref, dinv_ref, hprev_ref, h0_ref, h_ref, g_ref):
    dv = dinv_ref[...]
    t = t_ref[0] + t_ref[1]
    hnext = 0.5 * (dv * t + dv * dv * hprev_ref[...]) + 0.5 * h0_ref[...]
    h_ref[...] = hnext
    g_ref[...] = dv * hnext


def _combine(t2, dinv, hprev, h0, block):
    n, d = hprev.shape
    grid = n // block
    return pl.pallas_call(
        _combine_body,
        grid=(grid,),
        in_specs=[
            pl.BlockSpec((NC, block, d), lambda i: (0, i, 0)),
            pl.BlockSpec((block, 1), lambda i: (i, 0)),
            pl.BlockSpec((block, d), lambda i: (i, 0)),
            pl.BlockSpec((block, d), lambda i: (i, 0)),
        ],
        out_specs=[
            pl.BlockSpec((block, d), lambda i: (i, 0)),
            pl.BlockSpec((block, d), lambda i: (i, 0)),
        ],
        out_shape=[
            jax.ShapeDtypeStruct((n, d), jnp.float32),
            jax.ShapeDtypeStruct((n, d), jnp.float32),
        ],
    )(t2, dinv, hprev, h0)


def _final_body(t_ref, dinv_ref, hprev_ref, h0_ref, gam_ref, bet_ref,
                wproj_ref, bproj_ref, out_ref):
    dv = dinv_ref[...]
    t = t_ref[0] + t_ref[1]
    h2 = 0.5 * (dv * t + dv * dv * hprev_ref[...]) + 0.5 * h0_ref[...]
    zn = h2 * gam_ref[...] + bet_ref[...]
    out_ref[...] = lax.dot_general(
        zn, wproj_ref[...], (((1,), (1,)), ((), ())),
        preferred_element_type=jnp.float32) + bproj_ref[...]


def _final(t2, dinv, hprev, h0, gamma_s, beta, Wproj, bproj, block):
    n, d = hprev.shape
    d_out = Wproj.shape[0]
    grid = n // block
    return pl.pallas_call(
        _final_body,
        grid=(grid,),
        in_specs=[
            pl.BlockSpec((NC, block, d), lambda i: (0, i, 0)),
            pl.BlockSpec((block, 1), lambda i: (i, 0)),
            pl.BlockSpec((block, d), lambda i: (i, 0)),
            pl.BlockSpec((block, d), lambda i: (i, 0)),
            pl.BlockSpec((1, d), lambda i: (0, 0)),
            pl.BlockSpec((1, d), lambda i: (0, 0)),
            pl.BlockSpec((d_out, d), lambda i: (0, 0)),
            pl.BlockSpec((1, d_out), lambda i: (0, 0)),
        ],
        out_specs=pl.BlockSpec((block, d_out), lambda i: (i, 0)),
        out_shape=jax.ShapeDtypeStruct((n, d_out), jnp.float32),
    )(t2, dinv, hprev, h0, gamma_s.reshape(1, -1), beta.reshape(1, -1),
      Wproj, bproj.reshape(1, -1))


def _sc_deg_kernel(n, epw):
    mesh = plsc.VectorSubcoreMesh(core_axis_name="c", subcore_axis_name="s")

    @functools.partial(
        pl.kernel,
        out_type=jax.ShapeDtypeStruct((NW * n,), jnp.float32),
        mesh=mesh,
        compiler_params=pltpu.CompilerParams(needs_layout_passes=False, use_tc_tiling_on_sc=False),
        scratch_types=[
            pltpu.VMEM((epw,), jnp.int32),
            pltpu.VMEM((epw,), jnp.float32),
            pltpu.VMEM((n,), jnp.float32),
        ],
    )
    def deg_kernel(dst_hbm, w_hbm, out_hbm, dstv, wv, acc):
        wid = lax.axis_index("s") * NC + lax.axis_index("c")
        pltpu.sync_copy(dst_hbm.at[pl.ds(wid * epw, epw)], dstv)
        pltpu.sync_copy(w_hbm.at[pl.ds(wid * epw, epw)], wv)

        zeros = jnp.zeros((LANES,), jnp.float32)

        @plsc.parallel_loop(0, n // LANES, 1, unroll=8)
        def _(i):
            acc[pl.ds(i * LANES, LANES)] = zeros

        def edge_body(i, _):
            idx = dstv[pl.ds(i * LANES, LANES)]
            wts = wv[pl.ds(i * LANES, LANES)]
            plsc.addupdate_scatter(acc, [idx], wts)
            return 0

        lax.fori_loop(0, epw // LANES, edge_body, 0)
        pltpu.sync_copy(acc, out_hbm.at[pl.ds(wid * n, n)])

    return deg_kernel


def _sc_prop_kernel(n_pad, d, k0, k1):
    mesh = plsc.VectorSubcoreMesh(core_axis_name="c", subcore_axis_name="s")
    rows = n_pad // NS
    kmax = max(k0, k1)

    @functools.partial(
        pl.kernel,
        out_type=jax.ShapeDtypeStruct((NC, n_pad, d), jnp.float32),
        mesh=mesh,
        compiler_params=pltpu.CompilerParams(needs_layout_passes=False, use_tc_tiling_on_sc=False),
        scratch_types=[
            pltpu.VMEM((kmax * CHUNK,), jnp.int32),
            pltpu.VMEM((kmax, CHUNK), jnp.int32),
            pltpu.VMEM((kmax * CHUNK,), jnp.float32),
            [pltpu.VMEM((CHUNK, d), jnp.bfloat16) for _ in range(DEPTH)],
            [pltpu.VMEM((CHUNK, d), jnp.float32) for _ in range(2)],
            pltpu.VMEM_SHARED((n_pad, d), jnp.float32),
            [pltpu.SemaphoreType.DMA for _ in range(DEPTH)],
            [pltpu.SemaphoreType.DMA for _ in range(2)],
        ],
    )
    def prop_kernel(g_hbm, src_hbm, dst3_hbm, w_hbm, zeros_hbm, out_hbm,
                    sidx, didx, wbuf, gbufs, fbufs, acc, semg, sems):
        c = lax.axis_index("c")
        s = lax.axis_index("s")
        gchunk0 = jnp.where(c == 0, s * k0, NS * k0 + s * k1)
        nch = jnp.where(c == 0, k0, k1)
        base = gchunk0 * CHUNK

        pltpu.sync_copy(src_hbm.at[pl.ds(base, kmax * CHUNK)], sidx)
        pltpu.sync_copy(w_hbm.at[pl.ds(base, kmax * CHUNK)], wbuf)
        pltpu.sync_copy(dst3_hbm.at[pl.ds(gchunk0, kmax)], didx)

        pltpu.sync_copy(zeros_hbm.at[pl.ds(s * rows, rows)],
                        acc.at[pl.ds(s * rows, rows)])
        plsc.subcore_barrier()

        def gather_start(ci, b):
            pltpu.async_copy(g_hbm.at[sidx.at[pl.ds(ci * CHUNK, CHUNK)]],
                             gbufs[b], semg[b])

        def gather_wait(b):
            pltpu.make_async_copy(g_hbm.at[sidx.at[pl.ds(0, CHUNK)]],
                                  gbufs[b], semg[b]).wait()

        def scatter_start(ci, p):
            pltpu.async_copy(fbufs[p], acc.at[didx.at[ci]], sems[p], add=True)

        def scatter_wait(p):
            pltpu.make_async_copy(fbufs[p], acc.at[didx.at[0]],
                                  sems[p]).wait()

        def scale(ci, b, p):
            gbuf = gbufs[b]
            fbuf = fbufs[p]
            cbase = ci * CHUNK
            mask_hi = jnp.int32(-65536)

            @plsc.parallel_loop(0, CHUNK, 1, unroll=8)
            def _(e):
                wvec = plsc.load_gather(
                    wbuf,
                    [jnp.full((LANES,), 0, jnp.int32) + (cbase + e)])
                for q2 in range(d // 32):
                    v = gbuf[e, pl.ds(q2 * 32, 32)]
                    iv = plsc.bitcast(v, jnp.int32)
                    ev = lax.bitcast_convert_type(
                        lax.shift_left(iv, 16), jnp.float32)
                    od = lax.bitcast_convert_type(
                        iv & mask_hi, jnp.float32)
                    fbuf[e, pl.ds(q2 * 32, LANES)] = ev * wvec
                    fbuf[e, pl.ds(q2 * 32 + LANES, LANES)] = od * wvec

        for b in range(DEPTH - 1):
            gather_start(b, b)

        nouter = (nch + DEPTH - 1) // DEPTH

        def outer_body(gidx, _):
            i0 = gidx * DEPTH
            for b in range(DEPTH):
                i = i0 + b
                bn = (b + DEPTH - 1) % DEPTH
                nxt = i + DEPTH - 1

                @pl.when(i < nch)
                def _():
                    gather_wait(b)

                    @pl.when(nxt < nch)
                    def _():
                        gather_start(nxt, bn)

                    for p in range(2):
                        if b % 2 == p:

                            @pl.when(i >= 2)
                            def _():
                                scatter_wait(p)

                            scale(i, b, p)
                            scatter_start(i, p)
            return 0

        lax.fori_loop(0, nouter, outer_body, 0)
        for p in range(2):
            scatter_wait(p)

        plsc.subcore_barrier()
        pltpu.sync_copy(acc.at[pl.ds(s * rows, rows)],
                        out_hbm.at[c, pl.ds(s * rows, rows)])

    return prop_kernel


def kernel(x, edge_index, edge_weight, Wp, bp, Wm, bm, gamma, beta, Wproj,
           bproj):
    n = x.shape[0]
    d = Wm.shape[0]
    e = edge_weight.shape[0]

    per = NW * CHUNK
    e_pad = ((e + per - 1) // per) * per
    tot_chunks = e_pad // CHUNK

    k1 = (tot_chunks // NS) * K1_NUM // (K1_NUM + K0_NUM)
    k0 = tot_chunks // NS - k1
    kmax = max(k0, k1)
    extra = (NS * k0 + (NS - 1) * k1 + kmax) - tot_chunks
    extra = max(extra, 0) * CHUNK
    pad = e_pad - e + extra
    src = jnp.concatenate([edge_index[0], jnp.zeros((pad,), jnp.int32)])
    dst = jnp.concatenate([edge_index[1], jnp.zeros((pad,), jnp.int32)])
    w = jnp.concatenate([edge_weight, jnp.zeros((pad,), jnp.float32)])
    epw = e_pad // NW

    n_pad = ((n + NS * 8 - 1) // (NS * 8)) * (NS * 8)
    zeros = jnp.zeros((n_pad, d), jnp.float32)

    block = 400

    degp = _sc_deg_kernel(n, epw)(dst, w).reshape(NW, n)
    hid, dinv, g1 = _mlp_deg(x, Wp, bp, Wm, bm, degp)

    prop = _sc_prop_kernel(n_pad, d, k0, k1)
    dst3 = dst.reshape(-1, CHUNK)
    t1 = prop(_interleave_bf16(g1), src, dst3, w, zeros)
    h1, g2 = _combine(t1, dinv, hid, hid, block)
    t2 = prop(_interleave_bf16(g2), src, dst3, w, zeros)

    gamma_s = gamma / jnp.sqrt(jnp.float32(1.0 + 1e-5))
    return _final(t2, dinv, h1, hid, gamma_s, beta, Wproj, bproj, block)


# speedup vs baseline: 1.0570x; 1.0570x over previous
import functools

import jax
import jax.numpy as jnp
from jax import lax
from jax.experimental import pallas as pl
from jax.experimental.pallas import tpu as pltpu
from jax.experimental.pallas import tpu_sc as plsc

NC = 2
NS = 16
NW = NC * NS
LANES = 16
CHUNK = 128
DEPTH = 6
K0_NUM = 94
K1_NUM = 64


def _mlp_deg_body(x_ref, wp_ref, bp_ref, wm_ref, bm_ref, degp_ref,
                  hid_ref, dinv_ref, g_ref):
    h = jnp.maximum(
        lax.dot_general(x_ref[...], wp_ref[...], (((1,), (1,)), ((), ())),
                        preferred_element_type=jnp.float32) + bp_ref[...],
        0.0)
    hid = lax.dot_general(
        h, wm_ref[...], (((1,), (1,)), ((), ())),
        preferred_element_type=jnp.float32) + bm_ref[...]
    hid_ref[...] = hid
    deg = 1.0 + jnp.sum(degp_ref[...], axis=0)
    deg_safe = jnp.where(deg > 0, deg, 1.0)
    dinv = jnp.where(deg > 0, lax.rsqrt(deg_safe), 0.0)
    dinv_ref[...] = dinv[:, None]
    g_ref[...] = dinv[:, None] * hid


def _interleave_bf16(g):
    n, d = g.shape
    g4 = g.reshape(n, d // 32, 2, 16)
    gi = jnp.stack([g4[:, :, 0, :], g4[:, :, 1, :]], axis=-1)
    return gi.reshape(n, d).astype(jnp.bfloat16)


def _mlp_deg(x, Wp, bp, Wm, bm, degp):
    n = x.shape[0]
    d = Wm.shape[0]
    return pl.pallas_call(
        _mlp_deg_body,
        out_shape=[
            jax.ShapeDtypeStruct((n, d), jnp.float32),
            jax.ShapeDtypeStruct((n, 1), jnp.float32),
            jax.ShapeDtypeStruct((n, d), jnp.float32),
        ],
    )(x, Wp, bp.reshape(1, -1), Wm, bm.reshape(1, -1), degp)


def _combine_body(t_ref, dinv_ref, hprev_ref, h0_ref, h_ref, g_ref):
    dv = dinv_ref[...]
    t = t_ref[0] + t_ref[1]
    hnext = 0.5 * (dv * t + dv * dv * hprev_ref[...]) + 0.5 * h0_ref[...]
    h_ref[...] = hnext
    g_ref[...] = dv * hnext


def _combine(t2, dinv, hprev, h0, block):
    n, d = hprev.shape
    grid = n // block
    return pl.pallas_call(
        _combine_body,
        grid=(grid,),
        in_specs=[
            pl.BlockSpec((NC, block, d), lambda i: (0, i, 0)),
            pl.BlockSpec((block, 1), lambda i: (i, 0)),
            pl.BlockSpec((block, d), lambda i: (i, 0)),
            pl.BlockSpec((block, d), lambda i: (i, 0)),
        ],
        out_specs=[
            pl.BlockSpec((block, d), lambda i: (i, 0)),
            pl.BlockSpec((block, d), lambda i: (i, 0)),
        ],
        out_shape=[
            jax.ShapeDtypeStruct((n, d), jnp.float32),
            jax.ShapeDtypeStruct((n, d), jnp.float32),
        ],
    )(t2, dinv, hprev, h0)


def _final_body(t_ref, dinv_ref, hprev_ref, h0_ref, gam_ref, bet_ref,
                wproj_ref, bproj_ref, out_ref):
    dv = dinv_ref[...]
    t = t_ref[0] + t_ref[1]
    h2 = 0.5 * (dv * t + dv * dv * hprev_ref[...]) + 0.5 * h0_ref[...]
    zn = h2 * gam_ref[...] + bet_ref[...]
    out_ref[...] = lax.dot_general(
        zn, wproj_ref[...], (((1,), (1,)), ((), ())),
        preferred_element_type=jnp.float32) + bproj_ref[...]


def _final(t2, dinv, hprev, h0, gamma_s, beta, Wproj, bproj, block):
    n, d = hprev.shape
    d_out = Wproj.shape[0]
    grid = n // block
    return pl.pallas_call(
        _final_body,
        grid=(grid,),
        in_specs=[
            pl.BlockSpec((NC, block, d), lambda i: (0, i, 0)),
            pl.BlockSpec((block, 1), lambda i: (i, 0)),
            pl.BlockSpec((block, d), lambda i: (i, 0)),
            pl.BlockSpec((block, d), lambda i: (i, 0)),
            pl.BlockSpec((1, d), lambda i: (0, 0)),
            pl.BlockSpec((1, d), lambda i: (0, 0)),
            pl.BlockSpec((d_out, d), lambda i: (0, 0)),
            pl.BlockSpec((1, d_out), lambda i: (0, 0)),
        ],
        out_specs=pl.BlockSpec((block, d_out), lambda i: (i, 0)),
        out_shape=jax.ShapeDtypeStruct((n, d_out), jnp.float32),
    )(t2, dinv, hprev, h0, gamma_s.reshape(1, -1), beta.reshape(1, -1),
      Wproj, bproj.reshape(1, -1))


def _sc_deg_kernel(n, epw):
    mesh = plsc.VectorSubcoreMesh(core_axis_name="c", subcore_axis_name="s")

    @functools.partial(
        pl.kernel,
        out_type=jax.ShapeDtypeStruct((NW * n,), jnp.float32),
        mesh=mesh,
        compiler_params=pltpu.CompilerParams(needs_layout_passes=False, use_tc_tiling_on_sc=False),
        scratch_types=[
            pltpu.VMEM((epw,), jnp.int32),
            pltpu.VMEM((epw,), jnp.float32),
            pltpu.VMEM((n,), jnp.float32),
        ],
    )
    def deg_kernel(dst_hbm, w_hbm, out_hbm, dstv, wv, acc):
        wid = lax.axis_index("s") * NC + lax.axis_index("c")
        pltpu.sync_copy(dst_hbm.at[pl.ds(wid * epw, epw)], dstv)
        pltpu.sync_copy(w_hbm.at[pl.ds(wid * epw, epw)], wv)

        zeros = jnp.zeros((LANES,), jnp.float32)

        @plsc.parallel_loop(0, n // LANES, 1, unroll=8)
        def _(i):
            acc[pl.ds(i * LANES, LANES)] = zeros

        def edge_body(i, _):
            idx = dstv[pl.ds(i * LANES, LANES)]
            wts = wv[pl.ds(i * LANES, LANES)]
            plsc.addupdate_scatter(acc, [idx], wts)
            return 0

        lax.fori_loop(0, epw // LANES, edge_body, 0)
        pltpu.sync_copy(acc, out_hbm.at[pl.ds(wid * n, n)])

    return deg_kernel


def _sc_prop_kernel(n_pad, d, k0, k1):
    mesh = plsc.VectorSubcoreMesh(core_axis_name="c", subcore_axis_name="s")
    rows = n_pad // NS
    kmax = max(k0, k1)

    @functools.partial(
        pl.kernel,
        out_type=jax.ShapeDtypeStruct((NC, n_pad, d), jnp.float32),
        mesh=mesh,
        compiler_params=pltpu.CompilerParams(needs_layout_passes=False, use_tc_tiling_on_sc=False),
        scratch_types=[
            pltpu.VMEM((kmax * CHUNK,), jnp.int32),
            pltpu.VMEM((kmax, CHUNK), jnp.int32),
            pltpu.VMEM((kmax * CHUNK,), jnp.float32),
            [pltpu.VMEM((CHUNK, d), jnp.bfloat16) for _ in range(DEPTH)],
            [pltpu.VMEM((CHUNK, d), jnp.float32) for _ in range(2)],
            pltpu.VMEM_SHARED((n_pad, d), jnp.float32),
            [pltpu.SemaphoreType.DMA for _ in range(DEPTH)],
            [pltpu.SemaphoreType.DMA for _ in range(2)],
        ],
    )
    def prop_kernel(g_hbm, src_hbm, dst3_hbm, w_hbm, zeros_hbm, out_hbm,
                    sidx, didx, wbuf, gbufs, fbufs, acc, semg, sems):
        c = lax.axis_index("c")
        s = lax.axis_index("s")
        gchunk0 = jnp.where(c == 0, s * k0, NS * k0 + s * k1)
        nch = jnp.where(c == 0, k0, k1)
        base = gchunk0 * CHUNK

        pltpu.sync_copy(src_hbm.at[pl.ds(base, kmax * CHUNK)], sidx)
        pltpu.sync_copy(w_hbm.at[pl.ds(base, kmax * CHUNK)], wbuf)
        pltpu.sync_copy(dst3_hbm.at[pl.ds(gchunk0, kmax)], didx)

        pltpu.sync_copy(zeros_hbm.at[pl.ds(s * rows, rows)],
                        acc.at[pl.ds(s * rows, rows)])
        plsc.subcore_barrier()

        def gather_start(ci, b):
            pltpu.async_copy(g_hbm.at[sidx.at[pl.ds(ci * CHUNK, CHUNK)]],
                             gbufs[b], semg[b])

        def gather_wait(b):
            pltpu.make_async_copy(g_hbm.at[sidx.at[pl.ds(0, CHUNK)]],
                                  gbufs[b], semg[b]).wait()

        def scatter_start(ci, p):
            pltpu.async_copy(fbufs[p], acc.at[didx.at[ci]], sems[p], add=True)

        def scatter_wait(p):
            pltpu.make_async_copy(fbufs[p], acc.at[didx.at[0]],
                                  sems[p]).wait()

        def scale(ci, b, p):
            gbuf = gbufs[b]
            fbuf = fbufs[p]
            cbase = ci * CHUNK
            mask_hi = jnp.int32(-65536)

            @plsc.parallel_loop(0, CHUNK, 1, unroll=8)
            def _(e):
                wvec = plsc.load_gather(
                    wbuf,
                    [jnp.full((LANES,), 0, jnp.int32) + (cbase + e)])
                for q2 in range(d // 32):
                    v = gbuf[e, pl.ds(q2 * 32, 32)]
                    iv = plsc.bitcast(v, jnp.int32)
                    ev = lax.bitcast_convert_type(
                        lax.shift_left(iv, 16), jnp.float32)
                    od = lax.bitcast_convert_type(
                        iv & mask_hi, jnp.float32)
                    fbuf[e, pl.ds(q2 * 32, LANES)] = ev * wvec
                    fbuf[e, pl.ds(q2 * 32 + LANES, LANES)] = od * wvec

        for b in range(DEPTH - 1):
            gather_start(b, b)

        nouter = (nch + DEPTH - 1) // DEPTH

        def outer_body(gidx, _):
            i0 = gidx * DEPTH
            for b in range(DEPTH):
                i = i0 + b
                bn = (b + DEPTH - 1) % DEPTH
                nxt = i + DEPTH - 1

                @pl.when(i < nch)
                def _():
                    gather_wait(b)

                    @pl.when(nxt < nch)
                    def _():
                        gather_start(nxt, bn)

                    for p in range(2):
                        if b % 2 == p:

                            @pl.when(i >= 2)
                            def _():
                                scatter_wait(p)

                            scale(i, b, p)
                            scatter_start(i, p)
            return 0

        lax.fori_loop(0, nouter, outer_body, 0)
        for p in range(2):
            scatter_wait(p)

        plsc.subcore_barrier()
        pltpu.sync_copy(acc.at[pl.ds(s * rows, rows)],
                        out_hbm.at[c, pl.ds(s * rows, rows)])

    return prop_kernel


def kernel(x, edge_index, edge_weight, Wp, bp, Wm, bm, gamma, beta, Wproj,
           bproj):
    n = x.shape[0]
    d = Wm.shape[0]
    e = edge_weight.shape[0]

    per = NW * CHUNK
    e_pad = ((e + per - 1) // per) * per
    tot_chunks = e_pad // CHUNK

    k1 = (tot_chunks // NS) * K1_NUM // (K1_NUM + K0_NUM)
    k0 = tot_chunks // NS - k1
    kmax = max(k0, k1)
    extra = (NS * k0 + (NS - 1) * k1 + kmax) - tot_chunks
    extra = max(extra, 0) * CHUNK
    pad = e_pad - e + extra
    src = jnp.concatenate([edge_index[0], jnp.zeros((pad,), jnp.int32)])
    dst = jnp.concatenate([edge_index[1], jnp.zeros((pad,), jnp.int32)])
    w = jnp.concatenate([edge_weight, jnp.zeros((pad,), jnp.float32)])
    epw = e_pad // NW

    n_pad = ((n + NS * 8 - 1) // (NS * 8)) * (NS * 8)
    zeros = jnp.zeros((n_pad, d), jnp.float32)

    block = 400

    degp = _sc_deg_kernel(n, epw)(dst, w).reshape(NW, n)
    hid, dinv, g1 = _mlp_deg(x, Wp, bp, Wm, bm, degp)

    prop = _sc_prop_kernel(n_pad, d, k0, k1)
    dst3 = dst.reshape(-1, CHUNK)
    t1 = prop(_interleave_bf16(g1), src, dst3, w, zeros)
    h1, g2 = _combine(t1, dinv, hid, hid, block)
    t2 = prop(_interleave_bf16(g2), src, dst3, w, zeros)

    gamma_s = gamma / jnp.sqrt(jnp.float32(1.0 + 1e-5))
    return _final(t2, dinv, h1, hid, gamma_s, beta, Wproj, bproj, block)
